# baseline (speedup 1.0000x reference)
import jax
import jax.numpy as jnp
from jax.experimental import pallas as pl

_KNN = 5
_BOUND_LAMBDA = 1.0
_MAX_STEPS = 100
_UNROLL = 10
_NEG_BIG = -3.0e38


def _softmax(x):
    m = jnp.max(x, axis=1, keepdims=True)
    e = jnp.exp(x - m)
    return e / jnp.sum(e, axis=1, keepdims=True)


def _step(W, unary, Y, i, oldE, done):
    pairwise = _BOUND_LAMBDA * jnp.dot(W, Y, preferred_element_type=jnp.float32)
    Ynew = _softmax(-unary + pairwise)
    E = jnp.sum(
        unary * Ynew
        - _BOUND_LAMBDA * pairwise * Ynew
        + Ynew * jnp.log(jnp.clip(Ynew, 1e-20, None))
    )
    active = jnp.logical_and(jnp.logical_not(done), i < _MAX_STEPS)
    newdone = jnp.logical_and(i > 1, jnp.abs(E - oldE) <= 1e-08 * jnp.abs(oldE))
    Y = jnp.where(active, Ynew, Y)
    oldE = jnp.where(active, E, oldE)
    done = jnp.where(active, newdone, done)
    i = jnp.where(active, jnp.int32(i + 1), i)
    return Y, i, oldE, done


def _unary_of(scores):
    return -jnp.log(scores + 1e-10)


def _pack_state(i, oldE, done):
    r = jax.lax.broadcasted_iota(jnp.int32, (8, 128), 0)
    c = jax.lax.broadcasted_iota(jnp.int32, (8, 128), 1)
    zero = jnp.zeros((8, 128), jnp.float32)
    on_row0 = r == 0
    out = jnp.where(jnp.logical_and(on_row0, c == 0), i.astype(jnp.float32), zero)
    out = jnp.where(jnp.logical_and(on_row0, c == 1), oldE, out)
    out = jnp.where(
        jnp.logical_and(on_row0, c == 2), done.astype(jnp.float32), out
    )
    return out


def _main_kernel(scores_ref, feats_ref, y_ref, w_ref, state_ref):
    f = feats_ref[:]
    n = jnp.sqrt(jnp.sum(f * f, axis=1, keepdims=True))
    f = f / jnp.clip(n, 1e-12, None)

    G = jax.lax.dot_general(
        f, f, (((1,), (1,)), ((), ())), preferred_element_type=jnp.float32
    )
    N = G.shape[0]
    row_ids = jax.lax.broadcasted_iota(jnp.int32, (N, N), 0)
    col_ids = jax.lax.broadcasted_iota(jnp.int32, (N, N), 1)
    g = jnp.where(row_ids == col_ids, _NEG_BIG, G)

    W = jnp.zeros((N, N), jnp.float32)
    for _ in range(_KNN):
        m = jnp.max(g, axis=1, keepdims=True)
        cand = jnp.where(g == m, col_ids, N)
        idx = jnp.min(cand, axis=1, keepdims=True)
        hit = col_ids == idx
        W = W + hit.astype(jnp.float32)
        g = jnp.where(hit, _NEG_BIG, g)
    w_ref[:] = W

    unary = _unary_of(scores_ref[:])
    Y = _softmax(-unary)

    i = jnp.int32(0)
    oldE = jnp.array(jnp.inf, dtype=jnp.float32)
    done = jnp.array(False)
    for _ in range(_UNROLL):
        Y, i, oldE, done = _step(W, unary, Y, i, oldE, done)

    y_ref[:] = Y
    state_ref[:] = _pack_state(i, oldE, done)


def _fallback_kernel(scores_ref, w_ref, y_ref, state_ref, yo_ref, so_ref):
    unary = _unary_of(scores_ref[:])
    st = state_ref[:]
    i = st[0, 0].astype(jnp.int32)
    oldE = st[0, 1]
    done = st[0, 2] > 0.5
    Y, i, oldE, done = _step(w_ref[:], unary, y_ref[:], i, oldE, done)
    yo_ref[:] = Y
    so_ref[:] = _pack_state(i, oldE, done)


def kernel(scores_raw, feats):
    B, C, H, Wd = scores_raw.shape
    scores = scores_raw.reshape(-1, H * Wd)
    f = feats.reshape(feats.shape[:-3] + (-1,))
    if f.shape[0] == 1:
        f = jnp.squeeze(f, 0)
    M, L = scores.shape

    Y, W, state = pl.pallas_call(
        _main_kernel,
        out_shape=[
            jax.ShapeDtypeStruct((M, L), jnp.float32),
            jax.ShapeDtypeStruct((M, M), jnp.float32),
            jax.ShapeDtypeStruct((8, 128), jnp.float32),
        ],
    )(scores, f)

    step_call = pl.pallas_call(
        _fallback_kernel,
        out_shape=[
            jax.ShapeDtypeStruct((M, L), jnp.float32),
            jax.ShapeDtypeStruct((8, 128), jnp.float32),
        ],
    )

    def cond_fn(carry):
        _, st = carry
        return jnp.logical_and(st[0, 0] < _MAX_STEPS, st[0, 2] < 0.5)

    def body_fn(carry):
        Yc, st = carry
        Yn, stn = step_call(scores, W, Yc, st)
        return (Yn, stn)

    Y, _ = jax.lax.while_loop(cond_fn, body_fn, (Y, state))
    return Y


# speedup vs baseline: 34.9909x; 34.9909x over previous
import jax
import jax.numpy as jnp
from jax.experimental import pallas as pl
from jax.experimental.pallas import tpu as pltpu

_KNN = 5
_BOUND_LAMBDA = 1.0
_MAX_STEPS = 100
_NEG_BIG = -3.0e38


def _lame_kernel(scores_ref, feats_ref, out_ref, w_ref, unary_ref, y_ref):
    f = feats_ref[:]
    n = jnp.sqrt(jnp.sum(f * f, axis=1, keepdims=True))
    f = f / jnp.clip(n, 1e-12, None)

    G = jax.lax.dot_general(
        f, f, (((1,), (1,)), ((), ())), preferred_element_type=jnp.float32
    )
    N = G.shape[0]
    row_ids = jax.lax.broadcasted_iota(jnp.int32, (N, N), 0)
    col_ids = jax.lax.broadcasted_iota(jnp.int32, (N, N), 1)
    g = jnp.where(row_ids == col_ids, _NEG_BIG, G)

    W = jnp.zeros((N, N), jnp.float32)
    for _ in range(_KNN):
        m = jnp.max(g, axis=1, keepdims=True)
        cand = jnp.where(g == m, col_ids, N)
        idx = jnp.min(cand, axis=1, keepdims=True)
        hit = col_ids == idx
        W = W + hit.astype(jnp.float32)
        g = jnp.where(hit, _NEG_BIG, g)
    w_ref[:] = W

    unary = -jnp.log(scores_ref[:] + 1e-10)
    unary_ref[:] = unary
    m0 = jnp.max(-unary, axis=1, keepdims=True)
    e0 = jnp.exp(-unary - m0)
    y_ref[:] = e0 / jnp.sum(e0, axis=1, keepdims=True)

    def cond_fn(state):
        i, _, done = state
        return jnp.logical_and(i < _MAX_STEPS, jnp.logical_not(done))

    def body_fn(state):
        i, oldE, _ = state
        z = _BOUND_LAMBDA * jnp.dot(
            w_ref[:], y_ref[:], preferred_element_type=jnp.float32
        ) - unary_ref[:]
        m = jnp.max(z, axis=1, keepdims=True)
        e = jnp.exp(z - m)
        s = jnp.sum(e, axis=1, keepdims=True)
        y_ref[:] = e / s
        E = -jnp.sum(m + jnp.log(s))
        done = jnp.logical_and(i > 1, jnp.abs(E - oldE) <= 1e-08 * jnp.abs(oldE))
        return (i + 1, E, done)

    state0 = (jnp.int32(0), jnp.array(jnp.inf, dtype=jnp.float32), jnp.array(False))
    jax.lax.while_loop(cond_fn, body_fn, state0)
    out_ref[:] = y_ref[:]


def kernel(scores_raw, feats):
    B, C, H, Wd = scores_raw.shape
    scores = scores_raw.reshape(-1, H * Wd)
    f = feats.reshape(feats.shape[:-3] + (-1,))
    if f.shape[0] == 1:
        f = jnp.squeeze(f, 0)
    M, L = scores.shape
    return pl.pallas_call(
        _lame_kernel,
        out_shape=jax.ShapeDtypeStruct((M, L), jnp.float32),
        scratch_shapes=[
            pltpu.VMEM((M, M), jnp.float32),
            pltpu.VMEM((M, L), jnp.float32),
            pltpu.VMEM((M, L), jnp.float32),
        ],
    )(scores, f)


# speedup vs baseline: 40.0823x; 1.1455x over previous
import jax
import jax.numpy as jnp
from jax.experimental import pallas as pl
from jax.experimental.pallas import tpu as pltpu

_KNN = 5
_BOUND_LAMBDA = 1.0
_MAX_STEPS = 100
_NEG_BIG = -3.0e38


def _lame_kernel(scores_ref, feats_ref, out_ref, w_ref, unary_ref, y_ref):
    f = feats_ref[:]
    n = jnp.sqrt(jnp.sum(f * f, axis=1, keepdims=True))
    f = f / jnp.clip(n, 1e-12, None)

    G = jax.lax.dot_general(
        f, f, (((1,), (1,)), ((), ())), preferred_element_type=jnp.float32
    )
    N = G.shape[0]
    row_ids = jax.lax.broadcasted_iota(jnp.int32, (N, N), 0)
    col_ids = jax.lax.broadcasted_iota(jnp.int32, (N, N), 1)
    g = jnp.where(row_ids == col_ids, _NEG_BIG, G)

    for _ in range(_KNN):
        idx = jnp.argmax(g, axis=1, keepdims=True)
        hit = col_ids == idx
        g = jnp.where(hit, _NEG_BIG, g)
    w_ref[:] = jnp.where(
        jnp.logical_and(g == _NEG_BIG, row_ids != col_ids), 1.0, 0.0
    )

    unary = -jnp.log(scores_ref[:] + 1e-10)
    unary_ref[:] = unary
    m0 = jnp.max(-unary, axis=1, keepdims=True)
    e0 = jnp.exp(-unary - m0)
    y_ref[:] = e0 / jnp.sum(e0, axis=1, keepdims=True)

    def cond_fn(state):
        i, _, done = state
        return jnp.logical_and(i < _MAX_STEPS, jnp.logical_not(done))

    def body_fn(state):
        i, oldE, _ = state
        z = _BOUND_LAMBDA * jnp.dot(
            w_ref[:], y_ref[:], preferred_element_type=jnp.float32
        ) - unary_ref[:]
        m = jnp.max(z, axis=1, keepdims=True)
        e = jnp.exp(z - m)
        s = jnp.sum(e, axis=1, keepdims=True)
        y_ref[:] = e / s
        E = -jnp.sum(m + jnp.log(s))
        done = jnp.logical_and(i > 1, jnp.abs(E - oldE) <= 1e-08 * jnp.abs(oldE))
        return (i + 1, E, done)

    state0 = (jnp.int32(0), jnp.array(jnp.inf, dtype=jnp.float32), jnp.array(False))
    jax.lax.while_loop(cond_fn, body_fn, state0)
    out_ref[:] = y_ref[:]


def kernel(scores_raw, feats):
    B, C, H, Wd = scores_raw.shape
    scores = scores_raw.reshape(-1, H * Wd)
    f = feats.reshape(feats.shape[:-3] + (-1,))
    if f.shape[0] == 1:
        f = jnp.squeeze(f, 0)
    M, L = scores.shape
    return pl.pallas_call(
        _lame_kernel,
        out_shape=jax.ShapeDtypeStruct((M, L), jnp.float32),
        scratch_shapes=[
            pltpu.VMEM((M, M), jnp.float32),
            pltpu.VMEM((M, L), jnp.float32),
            pltpu.VMEM((M, L), jnp.float32),
        ],
    )(scores, f)
